# baseline (speedup 1.0000x reference)
import jax
import jax.numpy as jnp
from jax.experimental import pallas as pl
from jax.experimental.pallas import tpu as pltpu


def _table_body(months_ref, ch_ref, pos_ref, month_ref, out_ref):
    ch = ch_ref[...]
    pe = jnp.broadcast_to(pos_ref[0], (3, 192))
    me = jnp.broadcast_to(month_ref[0], (3, 192))
    zero = jnp.zeros((3, 192), jnp.float32)
    out_ref[0, 0] = jnp.concatenate([ch, pe, me, zero], axis=-1)


def _add_body(tokens_ref, table_ref, out_ref):
    out_ref[...] = tokens_ref[...] + table_ref[...][None]


def kernel(tokens, timestamps, channel_embed, pos_embed, month_table):
    b, h, w, t, bs, d = tokens.shape
    n = d // 4
    months = timestamps[:, :, 1].astype(jnp.int32)
    pos3 = pos_embed.reshape(pos_embed.shape[0], 1, n)
    month3 = month_table.reshape(month_table.shape[0], 1, n)

    table_spec = pltpu.PrefetchScalarGridSpec(
        num_scalar_prefetch=1,
        grid=(b, t),
        in_specs=[
            pl.BlockSpec((bs, n), lambda i, j, m: (0, 0)),
            pl.BlockSpec((1, 1, n), lambda i, j, m: (j, 0, 0)),
            pl.BlockSpec((1, 1, n), lambda i, j, m: (m[i, j], 0, 0)),
        ],
        out_specs=pl.BlockSpec((1, 1, bs, d), lambda i, j, m: (i, j, 0, 0)),
    )
    table = pl.pallas_call(
        _table_body,
        grid_spec=table_spec,
        out_shape=jax.ShapeDtypeStruct((b, t, bs, d), jnp.float32),
    )(months, channel_embed, pos3, month3)
    table = table.reshape(b, t * bs, d)

    hw = h * w
    bw = 64
    tok = tokens.reshape(b, hw, t * bs, d)
    out = pl.pallas_call(
        _add_body,
        grid=(b, hw // bw),
        in_specs=[
            pl.BlockSpec((1, bw, t * bs, d), lambda i, j: (i, j, 0, 0)),
            pl.BlockSpec((1, t * bs, d), lambda i, j: (i, 0, 0)),
        ],
        out_specs=pl.BlockSpec((1, bw, t * bs, d), lambda i, j: (i, j, 0, 0)),
        out_shape=jax.ShapeDtypeStruct((b, hw, t * bs, d), tokens.dtype),
    )(tok, table)
    return out.reshape(tokens.shape)


# speedup vs baseline: 1.2694x; 1.2694x over previous
import jax
import jax.numpy as jnp
from jax.experimental import pallas as pl


def _body(tokens_ref, months_ref, ch_ref, pos_ref, month_ref, out_ref):
    t = 12
    mrow = months_ref[0]
    sel = (jax.lax.broadcasted_iota(jnp.int32, (t, t), 0) == mrow)
    month_e = jax.lax.dot_general(
        sel.astype(jnp.float32), month_ref[...],
        dimension_numbers=(((0,), (0,)), ((), ())),
        preferred_element_type=jnp.float32)
    ch = jnp.broadcast_to(ch_ref[...][None], (t, 3, 192))
    pe = jnp.broadcast_to(pos_ref[:t][:, None], (t, 3, 192))
    me = jnp.broadcast_to(month_e[:, None], (t, 3, 192))
    zero = jnp.zeros((t, 3, 192), jnp.float32)
    addend = jnp.concatenate([ch, pe, me, zero], axis=-1)
    out_ref[...] = tokens_ref[...] + addend[None, None, None]


def kernel(tokens, timestamps, channel_embed, pos_embed, month_table):
    b, h, w, t, bs, d = tokens.shape
    n = d // 4
    months = timestamps[:, :, 1].astype(jnp.int32).reshape(b, 1, t)
    hb = 2
    tok_spec = pl.BlockSpec((1, hb, w, t, bs, d), lambda i, j: (i, j, 0, 0, 0, 0))
    return pl.pallas_call(
        _body,
        grid=(b, h // hb),
        in_specs=[
            tok_spec,
            pl.BlockSpec((1, 1, t), lambda i, j: (i, 0, 0)),
            pl.BlockSpec((bs, n), lambda i, j: (0, 0)),
            pl.BlockSpec((pos_embed.shape[0], n), lambda i, j: (0, 0)),
            pl.BlockSpec((t, n), lambda i, j: (0, 0)),
        ],
        out_specs=tok_spec,
        out_shape=jax.ShapeDtypeStruct(tokens.shape, tokens.dtype),
    )(tokens, months, channel_embed, pos_embed, month_table)


# speedup vs baseline: 5.9912x; 4.7198x over previous
import jax
import jax.numpy as jnp
from jax.experimental import pallas as pl


def _body(tokens_ref, months_ref, ch_ref, pos_ref, month_ref, out_ref):
    t = 12
    mrow = months_ref[0]
    sel = (jax.lax.broadcasted_iota(jnp.int32, (t, t), 0) == mrow)
    month_e = jax.lax.dot_general(
        sel.astype(jnp.float32), month_ref[...],
        dimension_numbers=(((0,), (0,)), ((), ())),
        preferred_element_type=jnp.float32)
    ch = jnp.broadcast_to(ch_ref[...][None], (t, 3, 192))
    pe = jnp.broadcast_to(pos_ref[:t][:, None], (t, 3, 192))
    me = jnp.broadcast_to(month_e[:, None], (t, 3, 192))
    zero = jnp.zeros((t, 3, 192), jnp.float32)
    addend = jnp.concatenate([ch, pe, me, zero], axis=-1)
    out_ref[...] = tokens_ref[...] + addend[None, None, :, :, None, :]


def kernel(tokens, timestamps, channel_embed, pos_embed, month_table):
    b, h, w, t, bs, d = tokens.shape
    n = d // 4
    months = timestamps[:, :, 1].astype(jnp.int32).reshape(b, 1, t)
    tok = jnp.transpose(tokens, (0, 1, 3, 4, 2, 5))
    hb = 4
    tok_spec = pl.BlockSpec((1, hb, t, bs, w, d), lambda i, j: (i, j, 0, 0, 0, 0))
    out = pl.pallas_call(
        _body,
        grid=(b, h // hb),
        in_specs=[
            tok_spec,
            pl.BlockSpec((1, 1, t), lambda i, j: (i, 0, 0)),
            pl.BlockSpec((bs, n), lambda i, j: (0, 0)),
            pl.BlockSpec((pos_embed.shape[0], n), lambda i, j: (0, 0)),
            pl.BlockSpec((t, n), lambda i, j: (0, 0)),
        ],
        out_specs=tok_spec,
        out_shape=jax.ShapeDtypeStruct((b, h, t, bs, w, d), tokens.dtype),
    )(tok, months, channel_embed, pos_embed, month_table)
    return jnp.transpose(out, (0, 1, 4, 2, 3, 5))
